# baseline (speedup 1.0000x reference)
import functools

import jax
import jax.numpy as jnp
from jax import lax
from jax.experimental import pallas as pl
from jax.experimental.pallas import tpu as pltpu
from jax.experimental.pallas import tpu_sc as plsc

_OUT_CHUNK = 4096


def _make_rowgather(B, V, D, num_cores, num_subcores):
    nw = num_cores * num_subcores
    rows_per_w = D // nw
    mesh = plsc.VectorSubcoreMesh(core_axis_name="c", subcore_axis_name="s")
    n_chunks = B // _OUT_CHUNK

    VA = 50048
    VB = V - VA

    scratch = [
        pltpu.VMEM((VA,), jnp.float32),
        pltpu.VMEM((VB,), jnp.float32),
        pltpu.VMEM((B,), jnp.int32),
        pltpu.VMEM((2, _OUT_CHUNK), jnp.float32),
        pltpu.SemaphoreType.DMA,
        pltpu.SemaphoreType.DMA,
    ]

    @functools.partial(
        pl.kernel,
        mesh=mesh,
        out_type=jax.ShapeDtypeStruct((D, B), jnp.float32),
        compiler_params=pltpu.CompilerParams(needs_layout_passes=False),
        scratch_types=scratch,
    )
    def rowgather(
        idx_hbm, tableT_hbm, outT_hbm, rowa_v, rowb_v, idx_v, out_v, sem_in, sem_out
    ):
        wid = lax.axis_index("s") * num_cores + lax.axis_index("c")

        def _row_dma(d):
            return [
                pltpu.async_copy(tableT_hbm.at[d, pl.ds(0, VA)], rowa_v, sem_in),
                pltpu.async_copy(tableT_hbm.at[d, pl.ds(VA, VB)], rowb_v, sem_in),
            ]

        idx_cp = pltpu.async_copy(idx_hbm, idx_v, sem_in)
        row_cps = _row_dma(wid)
        idx_cp.wait()
        for cp in row_cps:
            cp.wait()
        for r in range(rows_per_w):
            d = wid + r * nw
            out_cps = [None, None]
            for chunk in range(n_chunks):
                cbase = chunk * _OUT_CHUNK
                buf = chunk % 2
                if out_cps[buf] is not None:
                    out_cps[buf].wait()

                @plsc.parallel_loop(0, _OUT_CHUNK, step=16, unroll=8)
                def body(j, cbase=cbase, buf=buf):
                    ids = idx_v[pl.ds(cbase + j, 16)]
                    m = ids < VA
                    ia = jnp.minimum(ids, VA - 1)
                    ib = jnp.maximum(ids - VA, 0)
                    ga = plsc.load_gather(rowa_v, [ia])
                    gb = plsc.load_gather(rowb_v, [ib])
                    out_v[buf, pl.ds(j, 16)] = jnp.where(m, ga, gb)

                out_cps[buf] = pltpu.async_copy(
                    out_v.at[buf], outT_hbm.at[d, pl.ds(cbase, _OUT_CHUNK)], sem_out
                )
            for cp in out_cps:
                cp.wait()
            if r + 1 < rows_per_w:
                for cp in _row_dma(d + nw):
                    cp.wait()

    return rowgather


def kernel(context_ids, context_embeddings, prototypes):
    B = context_ids.shape[0]
    V, D = context_embeddings.shape
    info = plsc.get_sparse_core_info()
    rowgather = _make_rowgather(B, V, D, info.num_cores, info.num_subcores)
    outT = rowgather(context_ids.astype(jnp.int32), context_embeddings.T)
    return outT.T


# speedup vs baseline: 1.1864x; 1.1864x over previous
import functools

import jax
import jax.numpy as jnp
from jax import lax
from jax.experimental import pallas as pl
from jax.experimental.pallas import tpu as pltpu
from jax.experimental.pallas import tpu_sc as plsc

_OUT_CHUNK = 4096


def _make_rowgather(B, V, D, num_cores, num_subcores):
    nw = num_cores * num_subcores
    rows_per_w = D // nw
    mesh = plsc.VectorSubcoreMesh(core_axis_name="c", subcore_axis_name="s")
    n_chunks = B // _OUT_CHUNK

    @functools.partial(
        pl.kernel,
        mesh=mesh,
        out_type=jax.ShapeDtypeStruct((D, B), jnp.float32),
        compiler_params=pltpu.CompilerParams(needs_layout_passes=False),
        scratch_types=[
            pltpu.VMEM((V,), jnp.float32),
            pltpu.VMEM((B,), jnp.int32),
            pltpu.VMEM((2, _OUT_CHUNK), jnp.float32),
            pltpu.SemaphoreType.DMA,
            pltpu.SemaphoreType.DMA,
        ],
    )
    def rowgather(idx_hbm, tableT_hbm, outT_hbm, row_v, idx_v, out_v, sem_in, sem_out):
        wid = lax.axis_index("s") * num_cores + lax.axis_index("c")
        idx_cp = pltpu.async_copy(idx_hbm, idx_v, sem_in)
        row_cp = pltpu.async_copy(tableT_hbm.at[wid], row_v, sem_in)
        idx_cp.wait()
        row_cp.wait()
        for r in range(rows_per_w):
            d = wid + r * nw
            out_cps = [None, None]
            for chunk in range(n_chunks):
                cbase = chunk * _OUT_CHUNK
                buf = chunk % 2
                if out_cps[buf] is not None:
                    out_cps[buf].wait()

                @plsc.parallel_loop(0, _OUT_CHUNK, step=16, unroll=8)
                def body(j, cbase=cbase, buf=buf):
                    ids = idx_v[pl.ds(cbase + j, 16)]
                    out_v[buf, pl.ds(j, 16)] = plsc.load_gather(row_v, [ids])

                out_cps[buf] = pltpu.async_copy(
                    out_v.at[buf], outT_hbm.at[d, pl.ds(cbase, _OUT_CHUNK)], sem_out
                )
            next_row_cp = None
            if r + 1 < rows_per_w:
                next_row_cp = pltpu.async_copy(
                    tableT_hbm.at[d + nw], row_v, sem_in
                )
            for cp in out_cps:
                cp.wait()
            if next_row_cp is not None:
                next_row_cp.wait()

    return rowgather


def kernel(context_ids, context_embeddings, prototypes):
    B = context_ids.shape[0]
    V, D = context_embeddings.shape
    info = plsc.get_sparse_core_info()
    rowgather = _make_rowgather(B, V, D, info.num_cores, info.num_subcores)
    outT = rowgather(context_ids.astype(jnp.int32), context_embeddings.T)
    return outT.T
